# baseline (speedup 1.0000x reference)
import functools

import jax
import jax.numpy as jnp
from jax import lax
from jax.experimental import pallas as pl
from jax.experimental.pallas import tpu as pltpu
from jax.experimental.pallas import tpu_sc as plsc

_N = 10000
_E = 320000
_D_IN = 128
_H1 = 64
_H2 = 16

_NC = 2
_NS = 16
_NW = _NC * _NS
_EPW = _E // _NW
_CH = 128
_NCHUNK = -(-_EPW // _CH)
_EPW_PAD = _NCHUNK * _CH
_ACC_ROWS = 10240
_ZROWS = _ACC_ROWS // _NS
_FDEG = 16

_mesh = plsc.VectorSubcoreMesh(core_axis_name="c", subcore_axis_name="s")


def _zero_rows(zbuf, nrows, ncols):
    zero = jnp.zeros((16,), jnp.float32)

    def body(r, carry):
        for c16 in range(ncols // 16):
            zbuf[r, pl.ds(c16 * 16, 16)] = zero
        return carry

    lax.fori_loop(0, nrows, body, 0)


@functools.partial(
    pl.kernel,
    out_type=jax.ShapeDtypeStruct((2, _NC, _ACC_ROWS, _FDEG), jnp.float32),
    mesh=_mesh,
    scratch_types=[
        pltpu.VMEM((_NCHUNK, _CH), jnp.int32),
        pltpu.VMEM((_NCHUNK, _CH), jnp.int32),
        pltpu.VMEM((_CH, _FDEG), jnp.float32),
        pltpu.VMEM((_CH, _FDEG), jnp.float32),
        pltpu.VMEM_SHARED((_ACC_ROWS, _FDEG), jnp.float32),
        pltpu.VMEM_SHARED((_ACC_ROWS, _FDEG), jnp.float32),
    ],
    compiler_params=pltpu.CompilerParams(use_tc_tiling_on_sc=False),
)
def _degree_kernel(srcs_hbm, dsts_hbm, out_hbm, idx_s, idx_d, ones, zbuf,
                   acc_a, acc_b):
    c = lax.axis_index("c")
    s = lax.axis_index("s")
    wid = s * _NC + c
    pltpu.sync_copy(srcs_hbm.at[wid], idx_s)
    pltpu.sync_copy(dsts_hbm.at[wid], idx_d)

    one = jnp.ones((16,), jnp.float32)

    def obody(r, carry):
        ones[r, pl.ds(0, 16)] = one
        return carry

    lax.fori_loop(0, _CH, obody, 0)
    _zero_rows(zbuf, _CH, _FDEG)
    for k in range(_ZROWS // _CH):
        pltpu.sync_copy(zbuf, acc_a.at[pl.ds(s * _ZROWS + k * _CH, _CH)])
        pltpu.sync_copy(zbuf, acc_b.at[pl.ds(s * _ZROWS + k * _CH, _CH)])
    plsc.subcore_barrier()

    def body(j, carry):
        pltpu.sync_copy(ones, acc_a.at[idx_s.at[j]], add=True)
        pltpu.sync_copy(ones, acc_b.at[idx_d.at[j]], add=True)
        return carry

    lax.fori_loop(0, _NCHUNK, body, 0)
    plsc.subcore_barrier()
    pltpu.sync_copy(acc_a.at[pl.ds(s * _ZROWS, _ZROWS)],
                    out_hbm.at[0, c, pl.ds(s * _ZROWS, _ZROWS)])
    pltpu.sync_copy(acc_b.at[pl.ds(s * _ZROWS, _ZROWS)],
                    out_hbm.at[1, c, pl.ds(s * _ZROWS, _ZROWS)])


def _make_seg_sum(F):
    @functools.partial(
        pl.kernel,
        out_type=jax.ShapeDtypeStruct((_NC, _ACC_ROWS, F), jnp.float32),
        mesh=_mesh,
        scratch_types=[
            pltpu.VMEM((_NCHUNK, _CH), jnp.int32),
            pltpu.VMEM((_NCHUNK, _CH), jnp.int32),
            pltpu.VMEM((2, _CH, F), jnp.float32),
            pltpu.VMEM((_CH, F), jnp.float32),
            pltpu.VMEM_SHARED((_ACC_ROWS, F), jnp.float32),
            pltpu.SemaphoreType.DMA,
            pltpu.SemaphoreType.DMA,
        ],
        compiler_params=pltpu.CompilerParams(use_tc_tiling_on_sc=False),
    )
    def seg_sum(table_hbm, srcs_hbm, dsts_hbm, out_hbm, idx_s, idx_d, rows,
                zbuf, acc, gsem0, gsem1):
        c = lax.axis_index("c")
        s = lax.axis_index("s")
        wid = s * _NC + c
        pltpu.sync_copy(srcs_hbm.at[wid], idx_s)
        pltpu.sync_copy(dsts_hbm.at[wid], idx_d)
        _zero_rows(zbuf, _CH, F)
        for k in range(_ZROWS // _CH):
            pltpu.sync_copy(zbuf, acc.at[pl.ds(s * _ZROWS + k * _CH, _CH)])
        plsc.subcore_barrier()

        pltpu.async_copy(table_hbm.at[idx_s.at[0]], rows.at[0], gsem0)

        def body(i, carry):
            j = 2 * i
            pltpu.async_copy(table_hbm.at[idx_s.at[j + 1]], rows.at[1], gsem1)
            pltpu.make_async_copy(table_hbm.at[idx_s.at[j]], rows.at[0],
                                  gsem0).wait()
            pltpu.sync_copy(rows.at[0], acc.at[idx_d.at[j]], add=True)
            pltpu.async_copy(table_hbm.at[idx_s.at[j + 2]], rows.at[0], gsem0)
            pltpu.make_async_copy(table_hbm.at[idx_s.at[j + 1]], rows.at[1],
                                  gsem1).wait()
            pltpu.sync_copy(rows.at[1], acc.at[idx_d.at[j + 1]], add=True)
            return carry

        lax.fori_loop(0, (_NCHUNK - 1) // 2, body, 0)
        pltpu.make_async_copy(table_hbm.at[idx_s.at[_NCHUNK - 1]], rows.at[0],
                              gsem0).wait()
        pltpu.sync_copy(rows.at[0], acc.at[idx_d.at[_NCHUNK - 1]], add=True)
        plsc.subcore_barrier()
        pltpu.sync_copy(acc.at[pl.ds(s * _ZROWS, _ZROWS)],
                        out_hbm.at[c, pl.ds(s * _ZROWS, _ZROWS)])

    return seg_sum


_seg_sum_64 = _make_seg_sum(_H1)
_seg_sum_32 = _make_seg_sum(2 * _H2)


_RB = 1000


def _tca_body(x_ref, w1_ref, deg_ref, xw_ref, no_ref, ni_ref):
    dout = deg_ref[0, 0, :, 0:1] + deg_ref[0, 1, :, 0:1]
    din = deg_ref[1, 0, :, 0:1] + deg_ref[1, 1, :, 0:1]
    no = lax.rsqrt(jnp.maximum(dout, 1.0))
    ni = lax.rsqrt(jnp.maximum(din, 1.0))
    xw = jnp.dot(x_ref[...], w1_ref[...], preferred_element_type=jnp.float32)
    xw_ref[...] = xw * no
    no_ref[...] = no
    ni_ref[...] = ni


def _tcb_body(agg_ref, ni_ref, no_ref, b1_ref, wcat_ref, qw_ref):
    agg = agg_ref[0] + agg_ref[1]
    h = jnp.maximum(agg * ni_ref[...] + b1_ref[...], 0.0)
    q = h * no_ref[...]
    qw_ref[...] = jnp.dot(q, wcat_ref[...], preferred_element_type=jnp.float32)


def _tcc_body(agg_ref, ni_ref, bcat_ref, noise_ref, z_ref):
    a = (agg_ref[0] + agg_ref[1]) * ni_ref[...] + bcat_ref[...]
    mean = a[:, :_H2]
    log_std = a[:, _H2:]
    z_ref[...] = mean + noise_ref[...] * jnp.exp(log_std)


_DBI = 512
_DBJ = _N


def _dec_body(zi_ref, zj_ref, out_ref):
    logits = lax.dot_general(zi_ref[...], zj_ref[...],
                             (((1,), (1,)), ((), ())),
                             preferred_element_type=jnp.float32)
    out_ref[...] = 0.5 * (1.0 + jnp.tanh(0.5 * logits))


def kernel(features, edge_index, W1, b1, W2, b2, W3, b3):
    src = edge_index[0].reshape(_NW, _EPW)
    dst = edge_index[1].reshape(_NW, _EPW)
    pad = ((0, 0), (0, _EPW_PAD - _EPW))
    src_deg = jnp.pad(src, pad, constant_values=_N).reshape(_NW, _NCHUNK, _CH)
    dst_deg = jnp.pad(dst, pad, constant_values=_N).reshape(_NW, _NCHUNK, _CH)
    src_seg = jnp.pad(src, pad, constant_values=0).reshape(_NW, _NCHUNK, _CH)

    deg = _degree_kernel(src_deg, dst_deg)

    xw, no, ni = pl.pallas_call(
        _tca_body,
        grid=(_N // _RB,),
        in_specs=[
            pl.BlockSpec((_RB, _D_IN), lambda i: (i, 0)),
            pl.BlockSpec((_D_IN, _H1), lambda i: (0, 0)),
            pl.BlockSpec((2, _NC, _RB, _FDEG), lambda i: (0, 0, i, 0)),
        ],
        out_specs=[
            pl.BlockSpec((_RB, _H1), lambda i: (i, 0)),
            pl.BlockSpec((_RB, 1), lambda i: (i, 0)),
            pl.BlockSpec((_RB, 1), lambda i: (i, 0)),
        ],
        out_shape=[
            jax.ShapeDtypeStruct((_N, _H1), jnp.float32),
            jax.ShapeDtypeStruct((_N, 1), jnp.float32),
            jax.ShapeDtypeStruct((_N, 1), jnp.float32),
        ],
    )(features, W1, deg)

    agg1 = _seg_sum_64(xw, src_seg, dst_deg)

    wcat = jnp.concatenate([W2, W3], axis=1)
    qw = pl.pallas_call(
        _tcb_body,
        grid=(_N // _RB,),
        in_specs=[
            pl.BlockSpec((_NC, _RB, _H1), lambda i: (0, i, 0)),
            pl.BlockSpec((_RB, 1), lambda i: (i, 0)),
            pl.BlockSpec((_RB, 1), lambda i: (i, 0)),
            pl.BlockSpec((1, _H1), lambda i: (0, 0)),
            pl.BlockSpec((_H1, 2 * _H2), lambda i: (0, 0)),
        ],
        out_specs=pl.BlockSpec((_RB, 2 * _H2), lambda i: (i, 0)),
        out_shape=jax.ShapeDtypeStruct((_N, 2 * _H2), jnp.float32),
    )(agg1, ni, no, b1.reshape(1, _H1), wcat)

    agg2 = _seg_sum_32(qw, src_seg, dst_deg)

    noise = jax.random.normal(jax.random.key(42), (_N, _H2), dtype=jnp.float32)
    bcat = jnp.concatenate([b2, b3]).reshape(1, 2 * _H2)
    z = pl.pallas_call(
        _tcc_body,
        grid=(_N // _RB,),
        in_specs=[
            pl.BlockSpec((_NC, _RB, 2 * _H2), lambda i: (0, i, 0)),
            pl.BlockSpec((_RB, 1), lambda i: (i, 0)),
            pl.BlockSpec((1, 2 * _H2), lambda i: (0, 0)),
            pl.BlockSpec((_RB, _H2), lambda i: (i, 0)),
        ],
        out_specs=pl.BlockSpec((_RB, _H2), lambda i: (i, 0)),
        out_shape=jax.ShapeDtypeStruct((_N, _H2), jnp.float32),
    )(agg2, ni, bcat, noise)

    gi = -(-_N // _DBI)
    gj = -(-_N // _DBJ)
    pre = pl.pallas_call(
        _dec_body,
        grid=(gi, gj),
        in_specs=[
            pl.BlockSpec((_DBI, _H2), lambda i, j: (i, 0)),
            pl.BlockSpec((_DBJ, _H2), lambda i, j: (j, 0)),
        ],
        out_specs=pl.BlockSpec((_DBI, _DBJ), lambda i, j: (i, j)),
        out_shape=jax.ShapeDtypeStruct((_N, _N), jnp.float32),
    )(z, z)

    return (pre, z)


# speedup vs baseline: 1.2405x; 1.2405x over previous
import functools

import jax
import jax.numpy as jnp
from jax import lax
from jax.experimental import pallas as pl
from jax.experimental.pallas import tpu as pltpu
from jax.experimental.pallas import tpu_sc as plsc

_N = 10000
_E = 320000
_D_IN = 128
_H1 = 64
_H2 = 16

_NC = 2
_NS = 16
_NW = _NC * _NS
_EPW = _E // _NW
_CH = 128
_NCHUNK = -(-_EPW // _CH)
_EPW_PAD = _NCHUNK * _CH
_ACC_ROWS = 10240
_ZROWS = _ACC_ROWS // _NS
_FDEG = 16

_mesh = plsc.VectorSubcoreMesh(core_axis_name="c", subcore_axis_name="s")


def _zero_rows(zbuf, nrows, ncols):
    zero = jnp.zeros((16,), jnp.float32)

    def body(r, carry):
        for c16 in range(ncols // 16):
            zbuf[r, pl.ds(c16 * 16, 16)] = zero
        return carry

    lax.fori_loop(0, nrows, body, 0)


@functools.partial(
    pl.kernel,
    out_type=jax.ShapeDtypeStruct((2, _NC, _ACC_ROWS, _FDEG), jnp.float32),
    mesh=_mesh,
    scratch_types=[
        pltpu.VMEM((_NCHUNK, _CH), jnp.int32),
        pltpu.VMEM((_NCHUNK, _CH), jnp.int32),
        pltpu.VMEM((_CH, _FDEG), jnp.float32),
        pltpu.VMEM((_CH, _FDEG), jnp.float32),
        pltpu.VMEM_SHARED((_ACC_ROWS, _FDEG), jnp.float32),
        pltpu.VMEM_SHARED((_ACC_ROWS, _FDEG), jnp.float32),
    ],
    compiler_params=pltpu.CompilerParams(use_tc_tiling_on_sc=False),
)
def _degree_kernel(srcs_hbm, dsts_hbm, out_hbm, idx_s, idx_d, ones, zbuf,
                   acc_a, acc_b):
    c = lax.axis_index("c")
    s = lax.axis_index("s")
    wid = s * _NC + c
    pltpu.sync_copy(srcs_hbm.at[wid], idx_s)
    pltpu.sync_copy(dsts_hbm.at[wid], idx_d)

    one = jnp.ones((16,), jnp.float32)

    def obody(r, carry):
        ones[r, pl.ds(0, 16)] = one
        return carry

    lax.fori_loop(0, _CH, obody, 0)
    _zero_rows(zbuf, _CH, _FDEG)
    for k in range(_ZROWS // _CH):
        pltpu.sync_copy(zbuf, acc_a.at[pl.ds(s * _ZROWS + k * _CH, _CH)])
        pltpu.sync_copy(zbuf, acc_b.at[pl.ds(s * _ZROWS + k * _CH, _CH)])
    plsc.subcore_barrier()

    def body(j, carry):
        pltpu.sync_copy(ones, acc_a.at[idx_s.at[j]], add=True)
        pltpu.sync_copy(ones, acc_b.at[idx_d.at[j]], add=True)
        return carry

    lax.fori_loop(0, _NCHUNK, body, 0)
    plsc.subcore_barrier()
    pltpu.sync_copy(acc_a.at[pl.ds(s * _ZROWS, _ZROWS)],
                    out_hbm.at[0, c, pl.ds(s * _ZROWS, _ZROWS)])
    pltpu.sync_copy(acc_b.at[pl.ds(s * _ZROWS, _ZROWS)],
                    out_hbm.at[1, c, pl.ds(s * _ZROWS, _ZROWS)])


def _make_seg_sum(F):
    @functools.partial(
        pl.kernel,
        out_type=jax.ShapeDtypeStruct((_NC, _ACC_ROWS, F), jnp.float32),
        mesh=_mesh,
        scratch_types=[
            pltpu.VMEM((_NCHUNK, _CH), jnp.int32),
            pltpu.VMEM((_NCHUNK, _CH), jnp.int32),
            pltpu.VMEM((2, _CH, F), jnp.float32),
            pltpu.VMEM((_CH, F), jnp.float32),
            pltpu.VMEM_SHARED((_ACC_ROWS, F), jnp.float32),
            pltpu.VMEM_SHARED((_N, F), jnp.float32),
            pltpu.SemaphoreType.DMA,
            pltpu.SemaphoreType.DMA,
        ],
        compiler_params=pltpu.CompilerParams(use_tc_tiling_on_sc=False),
    )
    def seg_sum(table_hbm, srcs_hbm, dsts_hbm, out_hbm, idx_s, idx_d, rows,
                zbuf, acc, tbl, gsem0, gsem1):
        c = lax.axis_index("c")
        s = lax.axis_index("s")
        wid = s * _NC + c
        pltpu.sync_copy(srcs_hbm.at[wid], idx_s)
        pltpu.sync_copy(dsts_hbm.at[wid], idx_d)
        @pl.when(s < 10)
        def _stage():
            pltpu.sync_copy(table_hbm.at[pl.ds(s * 1000, 1000)],
                            tbl.at[pl.ds(s * 1000, 1000)])

        _zero_rows(zbuf, _CH, F)
        for k in range(_ZROWS // _CH):
            pltpu.sync_copy(zbuf, acc.at[pl.ds(s * _ZROWS + k * _CH, _CH)])
        plsc.subcore_barrier()

        pltpu.async_copy(tbl.at[idx_s.at[0]], rows.at[0], gsem0)

        def body(i, carry):
            j = 2 * i
            pltpu.async_copy(tbl.at[idx_s.at[j + 1]], rows.at[1], gsem1)
            pltpu.make_async_copy(tbl.at[idx_s.at[j]], rows.at[0],
                                  gsem0).wait()
            pltpu.sync_copy(rows.at[0], acc.at[idx_d.at[j]], add=True)
            pltpu.async_copy(tbl.at[idx_s.at[j + 2]], rows.at[0], gsem0)
            pltpu.make_async_copy(tbl.at[idx_s.at[j + 1]], rows.at[1],
                                  gsem1).wait()
            pltpu.sync_copy(rows.at[1], acc.at[idx_d.at[j + 1]], add=True)
            return carry

        lax.fori_loop(0, (_NCHUNK - 1) // 2, body, 0)
        pltpu.make_async_copy(tbl.at[idx_s.at[_NCHUNK - 1]], rows.at[0],
                              gsem0).wait()
        pltpu.sync_copy(rows.at[0], acc.at[idx_d.at[_NCHUNK - 1]], add=True)
        plsc.subcore_barrier()
        pltpu.sync_copy(acc.at[pl.ds(s * _ZROWS, _ZROWS)],
                        out_hbm.at[c, pl.ds(s * _ZROWS, _ZROWS)])

    return seg_sum


_seg_sum_64 = _make_seg_sum(_H1)
_seg_sum_32 = _make_seg_sum(2 * _H2)


_RB = 1000


def _tca_body(x_ref, w1_ref, deg_ref, xw_ref, no_ref, ni_ref):
    dout = deg_ref[0, 0, :, 0:1] + deg_ref[0, 1, :, 0:1]
    din = deg_ref[1, 0, :, 0:1] + deg_ref[1, 1, :, 0:1]
    no = lax.rsqrt(jnp.maximum(dout, 1.0))
    ni = lax.rsqrt(jnp.maximum(din, 1.0))
    xw = jnp.dot(x_ref[...], w1_ref[...], preferred_element_type=jnp.float32)
    xw_ref[...] = xw * no
    no_ref[...] = no
    ni_ref[...] = ni


def _tcb_body(agg_ref, ni_ref, no_ref, b1_ref, wcat_ref, qw_ref):
    agg = agg_ref[0] + agg_ref[1]
    h = jnp.maximum(agg * ni_ref[...] + b1_ref[...], 0.0)
    q = h * no_ref[...]
    qw_ref[...] = jnp.dot(q, wcat_ref[...], preferred_element_type=jnp.float32)


def _tcc_body(agg_ref, ni_ref, bcat_ref, noise_ref, z_ref):
    a = (agg_ref[0] + agg_ref[1]) * ni_ref[...] + bcat_ref[...]
    mean = a[:, :_H2]
    log_std = a[:, _H2:]
    z_ref[...] = mean + noise_ref[...] * jnp.exp(log_std)


_DBI = 256
_DBJ = _N


def _dec_body(zi_ref, zj_ref, out_ref):
    logits = lax.dot_general(zi_ref[...], zj_ref[...],
                             (((1,), (1,)), ((), ())),
                             preferred_element_type=jnp.float32)
    out_ref[...] = 0.5 * (1.0 + jnp.tanh(0.5 * logits))


def kernel(features, edge_index, W1, b1, W2, b2, W3, b3):
    src = edge_index[0].reshape(_NW, _EPW)
    dst = edge_index[1].reshape(_NW, _EPW)
    pad = ((0, 0), (0, _EPW_PAD - _EPW))
    src_deg = jnp.pad(src, pad, constant_values=_N).reshape(_NW, _NCHUNK, _CH)
    dst_deg = jnp.pad(dst, pad, constant_values=_N).reshape(_NW, _NCHUNK, _CH)
    src_seg = jnp.pad(src, pad, constant_values=0).reshape(_NW, _NCHUNK, _CH)

    deg = _degree_kernel(src_deg, dst_deg)

    xw, no, ni = pl.pallas_call(
        _tca_body,
        grid=(_N // _RB,),
        in_specs=[
            pl.BlockSpec((_RB, _D_IN), lambda i: (i, 0)),
            pl.BlockSpec((_D_IN, _H1), lambda i: (0, 0)),
            pl.BlockSpec((2, _NC, _RB, _FDEG), lambda i: (0, 0, i, 0)),
        ],
        out_specs=[
            pl.BlockSpec((_RB, _H1), lambda i: (i, 0)),
            pl.BlockSpec((_RB, 1), lambda i: (i, 0)),
            pl.BlockSpec((_RB, 1), lambda i: (i, 0)),
        ],
        out_shape=[
            jax.ShapeDtypeStruct((_N, _H1), jnp.float32),
            jax.ShapeDtypeStruct((_N, 1), jnp.float32),
            jax.ShapeDtypeStruct((_N, 1), jnp.float32),
        ],
    )(features, W1, deg)

    agg1 = _seg_sum_64(xw, src_seg, dst_deg)

    wcat = jnp.concatenate([W2, W3], axis=1)
    qw = pl.pallas_call(
        _tcb_body,
        grid=(_N // _RB,),
        in_specs=[
            pl.BlockSpec((_NC, _RB, _H1), lambda i: (0, i, 0)),
            pl.BlockSpec((_RB, 1), lambda i: (i, 0)),
            pl.BlockSpec((_RB, 1), lambda i: (i, 0)),
            pl.BlockSpec((1, _H1), lambda i: (0, 0)),
            pl.BlockSpec((_H1, 2 * _H2), lambda i: (0, 0)),
        ],
        out_specs=pl.BlockSpec((_RB, 2 * _H2), lambda i: (i, 0)),
        out_shape=jax.ShapeDtypeStruct((_N, 2 * _H2), jnp.float32),
    )(agg1, ni, no, b1.reshape(1, _H1), wcat)

    agg2 = _seg_sum_32(qw, src_seg, dst_deg)

    noise = jax.random.normal(jax.random.key(42), (_N, _H2), dtype=jnp.float32)
    bcat = jnp.concatenate([b2, b3]).reshape(1, 2 * _H2)
    z = pl.pallas_call(
        _tcc_body,
        grid=(_N // _RB,),
        in_specs=[
            pl.BlockSpec((_NC, _RB, 2 * _H2), lambda i: (0, i, 0)),
            pl.BlockSpec((_RB, 1), lambda i: (i, 0)),
            pl.BlockSpec((1, 2 * _H2), lambda i: (0, 0)),
            pl.BlockSpec((_RB, _H2), lambda i: (i, 0)),
        ],
        out_specs=pl.BlockSpec((_RB, _H2), lambda i: (i, 0)),
        out_shape=jax.ShapeDtypeStruct((_N, _H2), jnp.float32),
    )(agg2, ni, bcat, noise)

    gi = -(-_N // _DBI)
    gj = -(-_N // _DBJ)
    pre = pl.pallas_call(
        _dec_body,
        grid=(gi, gj),
        in_specs=[
            pl.BlockSpec((_DBI, _H2), lambda i, j: (i, 0)),
            pl.BlockSpec((_DBJ, _H2), lambda i, j: (j, 0)),
        ],
        out_specs=pl.BlockSpec((_DBI, _DBJ), lambda i, j: (i, j)),
        out_shape=jax.ShapeDtypeStruct((_N, _N), jnp.float32),
    )(z, z)

    return (pre, z)
